# baseline (speedup 1.0000x reference)
import functools

import jax
import jax.numpy as jnp
from jax import lax
from jax.experimental import pallas as pl
from jax.experimental.pallas import tpu as pltpu
from jax.experimental.pallas import tpu_sc as plsc

N_EXPERTS = 8
TOP_K = 2
ALPHA = 0.001
TOKEN_BLOCK = 1024
CHUNKS = 4
NUM_CORES = 2
NUM_SUBCORES = 16
NUM_WORKERS = NUM_CORES * NUM_SUBCORES
LANES = 16


def _gate_block(x_ref, w_ref, p_ref, acc_out_ref, acc_ref):
    i = pl.program_id(0)
    nb = pl.num_programs(0)

    x = x_ref[...]
    w = w_ref[...]
    logits = lax.dot_general(
        w, x, (((1,), (1,)), ((), ())), preferred_element_type=jnp.float32
    )

    m = jnp.max(logits, axis=0, keepdims=True)
    e = jnp.exp(logits - m)
    p = e / jnp.sum(e, axis=0, keepdims=True)
    p_ref[0, :, :] = p

    iota = lax.broadcasted_iota(jnp.int32, p.shape, 0)
    m1 = jnp.max(p, axis=0, keepdims=True)
    idx1 = jnp.min(jnp.where(p == m1, iota, N_EXPERTS), axis=0, keepdims=True)
    is1 = iota == idx1
    p2 = jnp.where(is1, -1.0, p)
    m2 = jnp.max(p2, axis=0, keepdims=True)
    idx2 = jnp.min(jnp.where(p2 == m2, iota, N_EXPERTS), axis=0, keepdims=True)
    is2 = iota == idx2

    part = jnp.concatenate(
        [p, jnp.where(is1 | is2, 1.0, 0.0)], axis=0
    )

    @pl.when(i == 0)
    def _init():
        acc_ref[...] = part

    @pl.when(i > 0)
    def _acc():
        acc_ref[...] += part

    @pl.when(i == nb - 1)
    def _fin():
        acc = jnp.sum(acc_ref[...], axis=1, keepdims=True)
        acc_out_ref[...] = jnp.broadcast_to(acc, (2 * N_EXPERTS, 128))


def _scores_kernel(hs, w):
    t, h = hs.shape
    tb = TOKEN_BLOCK
    nb = t // tb
    return pl.pallas_call(
        _gate_block,
        grid=(nb,),
        in_specs=[
            pl.BlockSpec((tb, h), lambda i: (i, 0)),
            pl.BlockSpec((N_EXPERTS, h), lambda i: (0, 0)),
        ],
        out_specs=[
            pl.BlockSpec((1, N_EXPERTS, tb), lambda i: (i, 0, 0)),
            pl.BlockSpec((2 * N_EXPERTS, 128), lambda i: (0, 0)),
        ],
        out_shape=[
            jax.ShapeDtypeStruct((nb, N_EXPERTS, tb), jnp.float32),
            jax.ShapeDtypeStruct((2 * N_EXPERTS, 128), jnp.float32),
        ],
        scratch_shapes=[pltpu.VMEM((2 * N_EXPERTS, tb), jnp.float32)],
        compiler_params=pltpu.CompilerParams(
            dimension_semantics=("arbitrary",),
        ),
    )(hs, w)


def _make_route_body(nb, tpw):
    per_block = TOKEN_BLOCK // tpw

    def _route_body(p_hbm, idx_hbm, wgt_hbm, p_v, idx_v, wgt_v):
        wid = lax.axis_index("s") * NUM_CORES + lax.axis_index("c")
        blk = wid // per_block
        off = (wid % per_block) * tpw
        pltpu.sync_copy(p_hbm.at[blk, :, pl.ds(off, tpw)], p_v)

        def chunk(j, carry):
            sl = pl.ds(j * LANES, LANES)
            pe = [p_v[e, sl] for e in range(N_EXPERTS)]

            best = pe[0]
            bi = jnp.zeros((LANES,), jnp.int32)
            for e in range(1, N_EXPERTS):
                upd = pe[e] > best
                best = jnp.where(upd, pe[e], best)
                bi = jnp.where(upd, e, bi)

            b2 = jnp.full((LANES,), -1.0, jnp.float32)
            bi2 = jnp.zeros((LANES,), jnp.int32)
            for e in range(N_EXPERTS):
                upd = (pe[e] > b2) & (bi != e)
                b2 = jnp.where(upd, pe[e], b2)
                bi2 = jnp.where(upd, e, bi2)

            inv = 1.0 / (best + b2 + 1e-20)
            idx_v[0, sl] = bi
            idx_v[1, sl] = bi2
            wgt_v[0, sl] = best * inv
            wgt_v[1, sl] = b2 * inv
            return carry

        lax.fori_loop(0, tpw // LANES, chunk, 0)
        pltpu.sync_copy(idx_v, idx_hbm.at[wid])
        pltpu.sync_copy(wgt_v, wgt_hbm.at[wid])

    return _route_body


def _route_sc(scores):
    nb = scores.shape[0]
    tpw = nb * TOKEN_BLOCK // NUM_WORKERS
    mesh = plsc.VectorSubcoreMesh(core_axis_name="c", subcore_axis_name="s")
    run = functools.partial(
        pl.kernel,
        mesh=mesh,
        out_type=[
            jax.ShapeDtypeStruct((NUM_WORKERS, TOP_K, tpw), jnp.int32),
            jax.ShapeDtypeStruct((NUM_WORKERS, TOP_K, tpw), jnp.float32),
        ],
        scratch_types=[
            pltpu.VMEM((N_EXPERTS, tpw), jnp.float32),
            pltpu.VMEM((TOP_K, tpw), jnp.int32),
            pltpu.VMEM((TOP_K, tpw), jnp.float32),
        ],
    )(_make_route_body(nb, tpw))
    return run(scores)


def kernel(hidden_states, kernel):
    bsz, seq_len, h = hidden_states.shape
    t = bsz * seq_len
    hs = hidden_states.reshape(t, h)
    tpc = t // CHUNKS

    idxs = []
    wgts = []
    accs = []
    for c in range(CHUNKS):
        scores, accp = _scores_kernel(hs[c * tpc : (c + 1) * tpc], kernel)
        idx3, wgt3 = _route_sc(scores)
        idxs.append(idx3.transpose(0, 2, 1).reshape(tpc, TOP_K))
        wgts.append(wgt3.transpose(0, 2, 1).reshape(tpc, TOP_K))
        accs.append(accp[:, 0])

    topk_idx = jnp.concatenate(idxs, axis=0)
    topk_weight = jnp.concatenate(wgts, axis=0)

    acc = sum(accs)
    pi = acc[:N_EXPERTS] / t
    fi = acc[N_EXPERTS:] * (N_EXPERTS / (t * TOP_K))
    aux_loss = jnp.sum(pi * fi) * ALPHA
    return (topk_idx, topk_weight, aux_loss)


# speedup vs baseline: 2.4254x; 2.4254x over previous
import functools

import jax
import jax.numpy as jnp
from jax import lax
from jax.experimental import pallas as pl
from jax.experimental.pallas import tpu as pltpu
from jax.experimental.pallas import tpu_sc as plsc

N_EXPERTS = 8
TOP_K = 2
ALPHA = 0.001
TOKEN_BLOCK = 1024
CHUNKS = 4
NUM_CORES = 2
NUM_SUBCORES = 16
NUM_WORKERS = NUM_CORES * NUM_SUBCORES
LANES = 16


def _gate_block(x_ref, w_ref, p_ref, acc_out_ref, acc_ref):
    i = pl.program_id(0)
    nb = pl.num_programs(0)

    x = x_ref[...]
    w = w_ref[...]
    logits = lax.dot_general(
        w, x, (((1,), (1,)), ((), ())), preferred_element_type=jnp.float32
    )

    m = jnp.max(logits, axis=0, keepdims=True)
    e = jnp.exp(logits - m)
    p = e / jnp.sum(e, axis=0, keepdims=True)
    p_ref[0, :, :] = p

    iota = lax.broadcasted_iota(jnp.int32, p.shape, 0)
    m1 = jnp.max(p, axis=0, keepdims=True)
    idx1 = jnp.min(jnp.where(p == m1, iota, N_EXPERTS), axis=0, keepdims=True)
    is1 = iota == idx1
    p2 = jnp.where(is1, -1.0, p)
    m2 = jnp.max(p2, axis=0, keepdims=True)
    idx2 = jnp.min(jnp.where(p2 == m2, iota, N_EXPERTS), axis=0, keepdims=True)
    is2 = iota == idx2

    part = jnp.concatenate(
        [p, jnp.where(is1 | is2, 1.0, 0.0)], axis=0
    )

    @pl.when(i == 0)
    def _init():
        acc_ref[...] = part

    @pl.when(i > 0)
    def _acc():
        acc_ref[...] += part

    @pl.when(i == nb - 1)
    def _fin():
        acc = jnp.sum(acc_ref[...], axis=1, keepdims=True)
        acc_out_ref[...] = jnp.broadcast_to(acc, (2 * N_EXPERTS, 128))


def _scores_kernel(hs, w, chunk, nb):
    t, h = hs.shape
    tb = TOKEN_BLOCK
    base = chunk * nb
    return pl.pallas_call(
        _gate_block,
        grid=(nb,),
        in_specs=[
            pl.BlockSpec((tb, h), lambda i: (base + i, 0)),
            pl.BlockSpec((N_EXPERTS, h), lambda i: (0, 0)),
        ],
        out_specs=[
            pl.BlockSpec((1, N_EXPERTS, tb), lambda i: (i, 0, 0)),
            pl.BlockSpec((2 * N_EXPERTS, 128), lambda i: (0, 0)),
        ],
        out_shape=[
            jax.ShapeDtypeStruct((nb, N_EXPERTS, tb), jnp.float32),
            jax.ShapeDtypeStruct((2 * N_EXPERTS, 128), jnp.float32),
        ],
        scratch_shapes=[pltpu.VMEM((2 * N_EXPERTS, tb), jnp.float32)],
        compiler_params=pltpu.CompilerParams(
            dimension_semantics=("arbitrary",),
        ),
    )(hs, w)


def _make_route_body(nb, tpw):
    per_block = TOKEN_BLOCK // tpw

    def _route_body(p_hbm, idx_hbm, wgt_hbm, p_v, idx_v, wgt_v):
        wid = lax.axis_index("s") * NUM_CORES + lax.axis_index("c")
        blk = wid // per_block
        off = (wid % per_block) * tpw
        pltpu.sync_copy(p_hbm.at[blk, :, pl.ds(off, tpw)], p_v)

        def chunk(j, carry):
            sl = pl.ds(j * LANES, LANES)
            pe = [p_v[e, sl] for e in range(N_EXPERTS)]

            best = pe[0]
            bi = jnp.zeros((LANES,), jnp.int32)
            for e in range(1, N_EXPERTS):
                upd = pe[e] > best
                best = jnp.where(upd, pe[e], best)
                bi = jnp.where(upd, e, bi)

            b2 = jnp.full((LANES,), -1.0, jnp.float32)
            bi2 = jnp.zeros((LANES,), jnp.int32)
            for e in range(N_EXPERTS):
                upd = (pe[e] > b2) & (bi != e)
                b2 = jnp.where(upd, pe[e], b2)
                bi2 = jnp.where(upd, e, bi2)

            inv = 1.0 / (best + b2 + 1e-20)
            idx_v[0, sl] = bi
            idx_v[1, sl] = bi2
            wgt_v[0, sl] = best * inv
            wgt_v[1, sl] = b2 * inv
            return carry

        lax.fori_loop(0, tpw // LANES, chunk, 0)
        pltpu.sync_copy(idx_v, idx_hbm.at[wid])
        pltpu.sync_copy(wgt_v, wgt_hbm.at[wid])

    return _route_body


def _route_sc(scores):
    nb = scores.shape[0]
    tpw = nb * TOKEN_BLOCK // NUM_WORKERS
    mesh = plsc.VectorSubcoreMesh(core_axis_name="c", subcore_axis_name="s")
    run = functools.partial(
        pl.kernel,
        mesh=mesh,
        out_type=[
            jax.ShapeDtypeStruct((NUM_WORKERS, TOP_K, tpw), jnp.int32),
            jax.ShapeDtypeStruct((NUM_WORKERS, TOP_K, tpw), jnp.float32),
        ],
        scratch_types=[
            pltpu.VMEM((N_EXPERTS, tpw), jnp.float32),
            pltpu.VMEM((TOP_K, tpw), jnp.int32),
            pltpu.VMEM((TOP_K, tpw), jnp.float32),
        ],
    )(_make_route_body(nb, tpw))
    return run(scores)


def kernel(hidden_states, kernel):
    bsz, seq_len, h = hidden_states.shape
    t = bsz * seq_len
    hs = hidden_states.reshape(t, h)
    tpc = t // CHUNKS

    idxs = []
    wgts = []
    accs = []
    nb = tpc // TOKEN_BLOCK
    for c in range(CHUNKS):
        scores, accp = _scores_kernel(hs, kernel, c, nb)
        idx3, wgt3 = _route_sc(scores)
        idxs.append(idx3.transpose(0, 2, 1).reshape(tpc, TOP_K))
        wgts.append(wgt3.transpose(0, 2, 1).reshape(tpc, TOP_K))
        accs.append(accp[:, 0])

    topk_idx = jnp.concatenate(idxs, axis=0)
    topk_weight = jnp.concatenate(wgts, axis=0)

    acc = sum(accs)
    pi = acc[:N_EXPERTS] / t
    fi = acc[N_EXPERTS:] * (N_EXPERTS / (t * TOP_K))
    aux_loss = jnp.sum(pi * fi) * ALPHA
    return (topk_idx, topk_weight, aux_loss)


# speedup vs baseline: 2.6028x; 1.0732x over previous
import functools

import jax
import jax.numpy as jnp
from jax import lax
from jax.experimental import pallas as pl
from jax.experimental.pallas import tpu as pltpu
from jax.experimental.pallas import tpu_sc as plsc

N_EXPERTS = 8
TOP_K = 2
ALPHA = 0.001
TOKEN_BLOCK = 1024
NUM_CORES = 2
NUM_SUBCORES = 16
NUM_WORKERS = NUM_CORES * NUM_SUBCORES
LANES = 16


def _gate_block(x_ref, w_ref, p_ref, aux_ref, acc_ref):
    i = pl.program_id(0)
    nb = pl.num_programs(0)

    x = x_ref[...]
    w = w_ref[...]
    logits = lax.dot_general(
        w, x, (((1,), (1,)), ((), ())), preferred_element_type=jnp.float32
    )

    m = jnp.max(logits, axis=0, keepdims=True)
    e = jnp.exp(logits - m)
    p = e / jnp.sum(e, axis=0, keepdims=True)
    p_ref[0, :, :] = p

    iota = lax.broadcasted_iota(jnp.int32, p.shape, 0)
    m1 = jnp.max(p, axis=0, keepdims=True)
    idx1 = jnp.min(jnp.where(p == m1, iota, N_EXPERTS), axis=0, keepdims=True)
    is1 = iota == idx1
    p2 = jnp.where(is1, -1.0, p)
    m2 = jnp.max(p2, axis=0, keepdims=True)
    idx2 = jnp.min(jnp.where(p2 == m2, iota, N_EXPERTS), axis=0, keepdims=True)
    is2 = iota == idx2

    part = jnp.concatenate(
        [p, jnp.where(is1 | is2, 1.0, 0.0)], axis=0
    )

    @pl.when(i == 0)
    def _init():
        acc_ref[...] = part

    @pl.when(i > 0)
    def _acc():
        acc_ref[...] += part

    @pl.when(i == nb - 1)
    def _fin():
        acc = jnp.sum(acc_ref[...], axis=1)
        total = nb * x.shape[0]
        pi = acc[:N_EXPERTS] / total
        fi = acc[N_EXPERTS:] * (N_EXPERTS / (total * TOP_K))
        aux = jnp.sum(pi * fi) * ALPHA
        aux_ref[...] = jnp.full((8, 128), aux, jnp.float32)


def _scores_kernel(hs, w):
    t, h = hs.shape
    tb = TOKEN_BLOCK
    nb = t // tb
    return pl.pallas_call(
        _gate_block,
        grid=(nb,),
        in_specs=[
            pl.BlockSpec((tb, h), lambda i: (i, 0)),
            pl.BlockSpec((N_EXPERTS, h), lambda i: (0, 0)),
        ],
        out_specs=[
            pl.BlockSpec((1, N_EXPERTS, tb), lambda i: (i, 0, 0)),
            pl.BlockSpec((8, 128), lambda i: (0, 0)),
        ],
        out_shape=[
            jax.ShapeDtypeStruct((nb, N_EXPERTS, tb), jnp.float32),
            jax.ShapeDtypeStruct((8, 128), jnp.float32),
        ],
        scratch_shapes=[pltpu.VMEM((2 * N_EXPERTS, tb), jnp.float32)],
        compiler_params=pltpu.CompilerParams(
            dimension_semantics=("arbitrary",),
        ),
    )(hs, w)


def _route_body(p_hbm, idx_hbm, wgt_hbm, p_v, idx_v, wgt_v):
    wid = lax.axis_index("s") * NUM_CORES + lax.axis_index("c")
    pltpu.sync_copy(p_hbm.at[wid], p_v)

    def chunk(j, carry):
        lane = lax.iota(jnp.int32, LANES)
        sl = pl.ds(j * LANES, LANES)
        pe = [p_v[e, sl] for e in range(N_EXPERTS)]

        best = pe[0]
        bi = jnp.zeros((LANES,), jnp.int32)
        for e in range(1, N_EXPERTS):
            upd = pe[e] > best
            best = jnp.where(upd, pe[e], best)
            bi = jnp.where(upd, e, bi)

        b2 = jnp.full((LANES,), -1.0, jnp.float32)
        bi2 = jnp.zeros((LANES,), jnp.int32)
        for e in range(N_EXPERTS):
            upd = (pe[e] > b2) & (bi != e)
            b2 = jnp.where(upd, pe[e], b2)
            bi2 = jnp.where(upd, e, bi2)

        inv = 1.0 / (best + b2 + 1e-20)
        idx_v[pl.ds(j * LANES, LANES)] = bi
        idx_v[pl.ds(TOKEN_BLOCK + j * LANES, LANES)] = bi2
        wgt_v[pl.ds(j * LANES, LANES)] = best * inv
        wgt_v[pl.ds(TOKEN_BLOCK + j * LANES, LANES)] = b2 * inv
        return carry

    lax.fori_loop(0, TOKEN_BLOCK // LANES, chunk, 0)
    pltpu.sync_copy(idx_v, idx_hbm.at[wid])
    pltpu.sync_copy(wgt_v, wgt_hbm.at[wid])


def _route_sc(scores):
    nw = scores.shape[0]
    tpw = scores.shape[2]
    mesh = plsc.VectorSubcoreMesh(core_axis_name="c", subcore_axis_name="s")
    run = functools.partial(
        pl.kernel,
        mesh=mesh,
        out_type=[
            jax.ShapeDtypeStruct((nw, TOP_K * tpw), jnp.int32),
            jax.ShapeDtypeStruct((nw, TOP_K * tpw), jnp.float32),
        ],
        scratch_types=[
            pltpu.VMEM((N_EXPERTS, tpw), jnp.float32),
            pltpu.VMEM((TOP_K * tpw,), jnp.int32),
            pltpu.VMEM((TOP_K * tpw,), jnp.float32),
        ],
    )(_route_body)
    return run(scores)


def kernel(hidden_states, kernel):
    bsz, seq_len, h = hidden_states.shape
    t = bsz * seq_len
    hs = hidden_states.reshape(t, h)

    scores, aux = _scores_kernel(hs, kernel)
    idx2d, wgt2d = _route_sc(scores)

    nw = idx2d.shape[0]
    tpw = idx2d.shape[1] // TOP_K
    topk_idx = idx2d.reshape(nw, TOP_K, tpw).transpose(0, 2, 1).reshape(t, TOP_K)
    topk_weight = wgt2d.reshape(nw, TOP_K, tpw).transpose(0, 2, 1).reshape(t, TOP_K)
    aux_loss = aux[0, 0]
    return (topk_idx, topk_weight, aux_loss)


# speedup vs baseline: 2.7530x; 1.0577x over previous
import functools

import jax
import jax.numpy as jnp
from jax import lax
from jax.experimental import pallas as pl
from jax.experimental.pallas import tpu as pltpu
from jax.experimental.pallas import tpu_sc as plsc

N_EXPERTS = 8
TOP_K = 2
ALPHA = 0.001
TOKEN_BLOCK = 1024
NUM_CORES = 2
NUM_SUBCORES = 16
NUM_WORKERS = NUM_CORES * NUM_SUBCORES
LANES = 16


def _gate_block(x_ref, w_ref, p_ref, aux_ref, acc_ref):
    i = pl.program_id(0)
    nb = pl.num_programs(0)

    x = x_ref[...]
    w = w_ref[...]
    logits = lax.dot_general(
        w, x, (((1,), (1,)), ((), ())), preferred_element_type=jnp.float32
    )

    m = jnp.max(logits, axis=0, keepdims=True)
    e = jnp.exp(logits - m)
    p = e / jnp.sum(e, axis=0, keepdims=True)
    p_ref[0, :, :] = p

    iota = lax.broadcasted_iota(jnp.int32, p.shape, 0)
    m1 = jnp.max(p, axis=0, keepdims=True)
    idx1 = jnp.min(jnp.where(p == m1, iota, N_EXPERTS), axis=0, keepdims=True)
    is1 = iota == idx1
    p2 = jnp.where(is1, -1.0, p)
    m2 = jnp.max(p2, axis=0, keepdims=True)
    idx2 = jnp.min(jnp.where(p2 == m2, iota, N_EXPERTS), axis=0, keepdims=True)
    is2 = iota == idx2

    part = jnp.concatenate(
        [p, jnp.where(is1 | is2, 1.0, 0.0)], axis=0
    )

    @pl.when(i == 0)
    def _init():
        acc_ref[...] = part

    @pl.when(i > 0)
    def _acc():
        acc_ref[...] += part

    @pl.when(i == nb - 1)
    def _fin():
        acc = jnp.sum(acc_ref[...], axis=1)
        total = nb * x.shape[0]
        pi = acc[:N_EXPERTS] / total
        fi = acc[N_EXPERTS:] * (N_EXPERTS / (total * TOP_K))
        aux = jnp.sum(pi * fi) * ALPHA
        aux_ref[...] = jnp.full((8, 128), aux, jnp.float32)


def _scores_kernel(hs, w):
    t, h = hs.shape
    tb = TOKEN_BLOCK
    nb = t // tb
    return pl.pallas_call(
        _gate_block,
        grid=(nb,),
        in_specs=[
            pl.BlockSpec((tb, h), lambda i: (i, 0)),
            pl.BlockSpec((N_EXPERTS, h), lambda i: (0, 0)),
        ],
        out_specs=[
            pl.BlockSpec((1, N_EXPERTS, tb), lambda i: (i, 0, 0)),
            pl.BlockSpec((8, 128), lambda i: (0, 0)),
        ],
        out_shape=[
            jax.ShapeDtypeStruct((nb, N_EXPERTS, tb), jnp.float32),
            jax.ShapeDtypeStruct((8, 128), jnp.float32),
        ],
        scratch_shapes=[pltpu.VMEM((2 * N_EXPERTS, tb), jnp.float32)],
        compiler_params=pltpu.CompilerParams(
            dimension_semantics=("arbitrary",),
        ),
    )(hs, w)


def _route_body(p_hbm, idx_hbm, wgt_hbm, p_v, idx_v, wgt_v):
    wid = lax.axis_index("s") * NUM_CORES + lax.axis_index("c")
    pltpu.sync_copy(p_hbm.at[wid], p_v)

    @plsc.parallel_loop(0, TOKEN_BLOCK // LANES, unroll=4)
    def chunk(j):
        sl = pl.ds(j * LANES, LANES)
        pe = [p_v[e, sl] for e in range(N_EXPERTS)]

        best = pe[0]
        bi = jnp.zeros((LANES,), jnp.int32)
        for e in range(1, N_EXPERTS):
            upd = pe[e] > best
            best = jnp.where(upd, pe[e], best)
            bi = jnp.where(upd, e, bi)

        b2 = jnp.full((LANES,), -1.0, jnp.float32)
        bi2 = jnp.zeros((LANES,), jnp.int32)
        for e in range(N_EXPERTS):
            upd = (pe[e] > b2) & (bi != e)
            b2 = jnp.where(upd, pe[e], b2)
            bi2 = jnp.where(upd, e, bi2)

        inv = 1.0 / (best + b2 + 1e-20)
        sl = pl.ds(j * LANES, LANES)
        idx_v[0, sl] = bi
        idx_v[1, sl] = bi2
        wgt_v[0, sl] = best * inv
        wgt_v[1, sl] = b2 * inv

    pltpu.sync_copy(idx_v, idx_hbm.at[wid])
    pltpu.sync_copy(wgt_v, wgt_hbm.at[wid])


def _route_sc(scores):
    nw = scores.shape[0]
    tpw = scores.shape[2]
    mesh = plsc.VectorSubcoreMesh(core_axis_name="c", subcore_axis_name="s")
    run = functools.partial(
        pl.kernel,
        mesh=mesh,
        out_type=[
            jax.ShapeDtypeStruct((nw, TOP_K, tpw), jnp.int32),
            jax.ShapeDtypeStruct((nw, TOP_K, tpw), jnp.float32),
        ],
        scratch_types=[
            pltpu.VMEM((N_EXPERTS, tpw), jnp.float32),
            pltpu.VMEM((TOP_K, tpw), jnp.int32),
            pltpu.VMEM((TOP_K, tpw), jnp.float32),
        ],
    )(_route_body)
    return run(scores)


def kernel(hidden_states, kernel):
    bsz, seq_len, h = hidden_states.shape
    t = bsz * seq_len
    hs = hidden_states.reshape(t, h)

    scores, aux = _scores_kernel(hs, kernel)
    idx2d, wgt2d = _route_sc(scores)

    topk_idx = idx2d.transpose(0, 2, 1).reshape(t, TOP_K)
    topk_weight = wgt2d.transpose(0, 2, 1).reshape(t, TOP_K)
    aux_loss = aux[0, 0]
    return (topk_idx, topk_weight, aux_loss)


# speedup vs baseline: 2.7635x; 1.0038x over previous
import functools

import jax
import jax.numpy as jnp
from jax import lax
from jax.experimental import pallas as pl
from jax.experimental.pallas import tpu as pltpu
from jax.experimental.pallas import tpu_sc as plsc

N_EXPERTS = 8
TOP_K = 2
ALPHA = 0.001
TOKEN_BLOCK = 1024
NUM_CORES = 2
NUM_SUBCORES = 16
NUM_WORKERS = NUM_CORES * NUM_SUBCORES
LANES = 16


def _gate_block(x_ref, w_ref, p_ref, aux_ref, acc_ref):
    i = pl.program_id(0)
    nb = pl.num_programs(0)

    x = x_ref[...]
    w = w_ref[...]
    logits = lax.dot_general(
        w, x, (((1,), (1,)), ((), ())), preferred_element_type=jnp.float32
    )

    m = jnp.max(logits, axis=0, keepdims=True)
    e = jnp.exp(logits - m)
    p = e / jnp.sum(e, axis=0, keepdims=True)
    p_ref[0, :, :] = p

    iota = lax.broadcasted_iota(jnp.int32, p.shape, 0)
    m1 = jnp.max(p, axis=0, keepdims=True)
    idx1 = jnp.min(jnp.where(p == m1, iota, N_EXPERTS), axis=0, keepdims=True)
    is1 = iota == idx1
    p2 = jnp.where(is1, -1.0, p)
    m2 = jnp.max(p2, axis=0, keepdims=True)
    idx2 = jnp.min(jnp.where(p2 == m2, iota, N_EXPERTS), axis=0, keepdims=True)
    is2 = iota == idx2

    part = jnp.concatenate(
        [p, jnp.where(is1 | is2, 1.0, 0.0)], axis=0
    )

    @pl.when(i == 0)
    def _init():
        acc_ref[...] = part

    @pl.when(i > 0)
    def _acc():
        acc_ref[...] += part

    @pl.when(i == nb - 1)
    def _fin():
        acc = jnp.sum(acc_ref[...], axis=1)
        total = nb * x.shape[0]
        pi = acc[:N_EXPERTS] / total
        fi = acc[N_EXPERTS:] * (N_EXPERTS / (total * TOP_K))
        aux = jnp.sum(pi * fi) * ALPHA
        aux_ref[...] = jnp.full((8, 128), aux, jnp.float32)


def _scores_kernel(hs, w):
    t, h = hs.shape
    tb = TOKEN_BLOCK
    nb = t // tb
    return pl.pallas_call(
        _gate_block,
        grid=(nb,),
        in_specs=[
            pl.BlockSpec((tb, h), lambda i: (i, 0)),
            pl.BlockSpec((N_EXPERTS, h), lambda i: (0, 0)),
        ],
        out_specs=[
            pl.BlockSpec((1, N_EXPERTS, tb), lambda i: (i, 0, 0)),
            pl.BlockSpec((8, 128), lambda i: (0, 0)),
        ],
        out_shape=[
            jax.ShapeDtypeStruct((nb, N_EXPERTS, tb), jnp.float32),
            jax.ShapeDtypeStruct((8, 128), jnp.float32),
        ],
        scratch_shapes=[pltpu.VMEM((2 * N_EXPERTS, tb), jnp.float32)],
        compiler_params=pltpu.CompilerParams(
            dimension_semantics=("arbitrary",),
        ),
    )(hs, w)


def _route_body(p_hbm, idx_hbm, wgt_hbm, p_v, idx_v, wgt_v):
    wid = lax.axis_index("s") * NUM_CORES + lax.axis_index("c")
    pltpu.sync_copy(p_hbm.at[wid], p_v)

    @plsc.parallel_loop(0, TOKEN_BLOCK // LANES, unroll=8)
    def chunk(j):
        sl = pl.ds(j * LANES, LANES)
        pe = [p_v[e, sl] for e in range(N_EXPERTS)]

        best = pe[0]
        bi = jnp.zeros((LANES,), jnp.int32)
        for e in range(1, N_EXPERTS):
            upd = pe[e] > best
            best = jnp.where(upd, pe[e], best)
            bi = jnp.where(upd, e, bi)

        b2 = jnp.full((LANES,), -1.0, jnp.float32)
        bi2 = jnp.zeros((LANES,), jnp.int32)
        for e in range(N_EXPERTS):
            upd = (pe[e] > b2) & (bi != e)
            b2 = jnp.where(upd, pe[e], b2)
            bi2 = jnp.where(upd, e, bi2)

        inv = 1.0 / (best + b2 + 1e-20)
        sl = pl.ds(j * LANES, LANES)
        idx_v[0, sl] = bi
        idx_v[1, sl] = bi2
        wgt_v[0, sl] = best * inv
        wgt_v[1, sl] = b2 * inv

    pltpu.sync_copy(idx_v, idx_hbm.at[wid])
    pltpu.sync_copy(wgt_v, wgt_hbm.at[wid])


def _route_sc(scores):
    nw = scores.shape[0]
    tpw = scores.shape[2]
    mesh = plsc.VectorSubcoreMesh(core_axis_name="c", subcore_axis_name="s")
    run = functools.partial(
        pl.kernel,
        mesh=mesh,
        out_type=[
            jax.ShapeDtypeStruct((nw, TOP_K, tpw), jnp.int32),
            jax.ShapeDtypeStruct((nw, TOP_K, tpw), jnp.float32),
        ],
        scratch_types=[
            pltpu.VMEM((N_EXPERTS, tpw), jnp.float32),
            pltpu.VMEM((TOP_K, tpw), jnp.int32),
            pltpu.VMEM((TOP_K, tpw), jnp.float32),
        ],
    )(_route_body)
    return run(scores)


def kernel(hidden_states, kernel):
    bsz, seq_len, h = hidden_states.shape
    t = bsz * seq_len
    hs = hidden_states.reshape(t, h)

    scores, aux = _scores_kernel(hs, kernel)
    idx2d, wgt2d = _route_sc(scores)

    topk_idx = idx2d.transpose(0, 2, 1).reshape(t, TOP_K)
    topk_weight = wgt2d.transpose(0, 2, 1).reshape(t, TOP_K)
    aux_loss = aux[0, 0]
    return (topk_idx, topk_weight, aux_loss)
